# Initial kernel scaffold; baseline (speedup 1.0000x reference)
import jax
import jax.numpy as jnp
from jax.experimental import pallas as pl


def kernel(x, edge_index_d0, edge_index_d1, batch, ogt, W0_d0, W0_d1, Wc0, bc0, W1_d0, W1_d1, Wc1, bc1, ogt_W1, ogt_b1, ogt_W2, ogt_b2, fc_W1, fc_b1, fc_W2, fc_b2):
    raise NotImplementedError("write your pallas kernel here")


# speedup vs baseline: 3.9113x; 3.9113x over previous
import functools

import jax
import jax.numpy as jnp
from jax import lax
from jax.experimental import pallas as pl
from jax.experimental.pallas import tpu as pltpu
from jax.experimental.pallas import tpu_sc as plsc

N = 10000
E = 160000
D = 256
HID = 256
G = 64

RB = 400
NBLK = N // RB

CUT = 5200
ACCR = 5632
ZR = ACCR // 16
CH = 80
NCH = (E // 16) // CH


def _lrelu(v):
    return jnp.where(v >= 0, v, 0.01 * v)


def _dot(a, b):
    return lax.dot_general(a, b, (((1,), (0,)), ((), ())),
                           precision=lax.Precision.HIGHEST,
                           preferred_element_type=jnp.float32)


def _dot0(a, b):
    return lax.dot_general(a, b, (((0,), (0,)), ((), ())),
                           precision=lax.Precision.HIGHEST,
                           preferred_element_type=jnp.float32)


def _deg_body(dstb_h, zdeg_h, out_h, dd, deg2, tmp, acc, degs):
    c = lax.axis_index("c")
    s = lax.axis_index("s")

    pltpu.sync_copy(zdeg_h, deg2)

    eoff = c * E + s * (E // 16)
    ones16 = jnp.ones((16,), jnp.float32)

    def chunk(k, carry):
        pltpu.sync_copy(dstb_h.at[pl.ds(eoff + CH * k, CH)], dd)
        for j in range(CH // 16):
            dv = dd[pl.ds(16 * j, 16)]
            row = lax.shift_right_logical(dv, jnp.int32(7))
            col = jnp.bitwise_and(dv, jnp.int32(127))
            plsc.addupdate_scatter(deg2, [row, col], ones16)
        return carry

    lax.fori_loop(jnp.int32(0), jnp.int32(NCH), chunk, jnp.int32(0))

    pltpu.sync_copy(deg2, degs.at[s])
    plsc.subcore_barrier()

    for i in range(16):
        pltpu.sync_copy(degs.at[jnp.int32(i), pl.ds(8 * s, 8)],
                        tmp.at[jnp.int32(i)])
    for r in range(8):
        for j in range(8):
            v = tmp[jnp.int32(0), jnp.int32(r), pl.ds(16 * j, 16)]
            for i in range(1, 16):
                v = v + tmp[jnp.int32(i), jnp.int32(r), pl.ds(16 * j, 16)]
            acc[jnp.int32(r), pl.ds(16 * j, 16)] = v
    pltpu.sync_copy(acc, out_h.at[c, s])


def _scat_body(u0_h, u1_h, srcb_h, dstb_h, z_h, o0_h, o1_h,
               sidx, didx, dloc, bufa, bufb, acca, accb):
    c = lax.axis_index("c")
    s = lax.axis_index("s")
    i0 = jnp.int32(0)
    base = c * CUT
    size = CUT - 400 * c
    iota16 = lax.broadcasted_iota(jnp.int32, (16,), 0)
    trash = size + s * 16 + iota16
    HH = HID // 2

    for d, (u_h, o_h) in enumerate(((u0_h, o0_h), (u1_h, o1_h))):
        pltpu.sync_copy(z_h, acca.at[pl.ds(ZR * s, ZR)])
        pltpu.sync_copy(z_h, accb.at[pl.ds(ZR * s, ZR)])
        plsc.subcore_barrier()

        eoff = d * E + s * (E // 16)

        def chunk(k, carry):
            pltpu.sync_copy(srcb_h.at[pl.ds(eoff + CH * k, CH)], sidx.at[i0])
            pltpu.sync_copy(dstb_h.at[pl.ds(eoff + CH * k, CH)], didx.at[i0])
            for j in range(CH // 16):
                dv = didx[i0, pl.ds(16 * j, 16)]
                inr = jnp.logical_and(dv >= base, dv < base + size)
                dloc[i0, pl.ds(16 * j, 16)] = jnp.where(inr, dv - base, trash)
            il = sidx.at[i0]
            pltpu.sync_copy(u_h.at[il, pl.ds(0, HH)], bufa)
            pltpu.sync_copy(u_h.at[il, pl.ds(HH, HH)], bufb)
            ol = dloc.at[i0]
            pltpu.sync_copy(bufa, acca.at[ol], add=True)
            pltpu.sync_copy(bufb, accb.at[ol], add=True)
            return carry

        lax.fori_loop(jnp.int32(0), jnp.int32(NCH), chunk, jnp.int32(0))
        plsc.subcore_barrier()

        @pl.when(jnp.logical_and(c == 0, s < 13))
        def _():
            pltpu.sync_copy(acca.at[pl.ds(400 * s, 400)],
                            o_h.at[pl.ds(400 * s, 400), pl.ds(0, HH)])
            pltpu.sync_copy(accb.at[pl.ds(400 * s, 400)],
                            o_h.at[pl.ds(400 * s, 400), pl.ds(HH, HH)])

        @pl.when(jnp.logical_and(c == 1, s < 12))
        def _():
            pltpu.sync_copy(acca.at[pl.ds(400 * s, 400)],
                            o_h.at[pl.ds(CUT + 400 * s, 400), pl.ds(0, HH)])
            pltpu.sync_copy(accb.at[pl.ds(400 * s, 400)],
                            o_h.at[pl.ds(CUT + 400 * s, 400), pl.ds(HH, HH)])

        plsc.subcore_barrier()


def _sc_degrees(dst_both, zdeg):
    mesh = plsc.VectorSubcoreMesh(core_axis_name="c", subcore_axis_name="s")
    f = functools.partial(
        pl.kernel, mesh=mesh,
        compiler_params=pltpu.CompilerParams(needs_layout_passes=False),
        out_type=jax.ShapeDtypeStruct((2, 16, 8, 128), jnp.float32),
        scratch_types=[
            pltpu.VMEM((CH,), jnp.int32),
            pltpu.VMEM((128, 128), jnp.float32),
            pltpu.VMEM((16, 8, 128), jnp.float32),
            pltpu.VMEM((8, 128), jnp.float32),
            pltpu.VMEM_SHARED((16, 128, 128), jnp.float32),
        ],
    )(_deg_body)
    return f(dst_both, zdeg)


def _sc_scatter(u0, u1, src_both, dst_both, zrows):
    mesh = plsc.VectorSubcoreMesh(core_axis_name="c", subcore_axis_name="s")
    f = functools.partial(
        pl.kernel, mesh=mesh,
        compiler_params=pltpu.CompilerParams(needs_layout_passes=False),
        out_type=(jax.ShapeDtypeStruct((N, HID), jnp.float32),
                  jax.ShapeDtypeStruct((N, HID), jnp.float32)),
        scratch_types=[
            pltpu.VMEM((1, CH), jnp.int32),
            pltpu.VMEM((1, CH), jnp.int32),
            pltpu.VMEM((1, CH), jnp.int32),
            pltpu.VMEM((CH, HID // 2), jnp.float32),
            pltpu.VMEM((CH, HID // 2), jnp.float32),
            pltpu.VMEM_SHARED((ACCR, HID // 2), jnp.float32),
            pltpu.VMEM_SHARED((ACCR, HID // 2), jnp.float32),
        ],
    )(_scat_body)
    return f(u0, u1, src_both, dst_both, zrows)


def _stage1_body(cnt0_r, cnt1_r, x_r, w0_r, w1_r, u0_r, u1_r):
    xb = x_r[...]
    d0 = lax.rsqrt(cnt0_r[0, 0, :] + 1.0)
    d1 = lax.rsqrt(cnt1_r[0, 0, :] + 1.0)
    u0_r[...] = _dot(xb, w0_r[...]) * d0[:, None]
    u1_r[...] = _dot(xb, w1_r[...]) * d1[:, None]


def _stage2_body(cnt0_r, cnt1_r, s0_r, s1_r, u0_r, u1_r, wc_r, bc_r, h_r):
    d0 = lax.rsqrt(cnt0_r[0, 0, :] + 1.0)
    d1 = lax.rsqrt(cnt1_r[0, 0, :] + 1.0)
    m = 0.5 * (d0[:, None] * (s0_r[...] + u0_r[...])
               + d1[:, None] * (s1_r[...] + u1_r[...]))
    h_r[...] = _lrelu(_dot(m, wc_r[...]) + bc_r[...])


def _pool_body(batch_r, h_r, ogt_r, ow1_r, ob1_r, ow2_r, ob2_r,
               fa_r, fb_r, fb1_r, fw2_r, fb2_r, out_r, pooled, cnts):
    i = pl.program_id(0)

    @pl.when(i == 0)
    def _():
        pooled[...] = jnp.zeros((G, HID), jnp.float32)
        cnts[...] = jnp.zeros((1, G), jnp.float32)

    b = batch_r[0, 0, :]
    oh = (b[:, None] == lax.broadcasted_iota(jnp.int32, (RB, G), 1)
          ).astype(jnp.float32)
    pooled[...] += _dot0(oh, h_r[...])
    cnts[...] += jnp.sum(oh, axis=0)[None, :]

    @pl.when(i == NBLK - 1)
    def _():
        cc = jnp.maximum(cnts[0, :], 1.0)
        gemb = pooled[...] / cc[:, None]
        o1 = _lrelu(_dot(ogt_r[...], ow1_r[...]) + ob1_r[...])
        o2 = _lrelu(_dot(o1, ow2_r[...]) + ob2_r[...])
        z1 = _lrelu(_dot(gemb, fa_r[...]) + _dot(o2, fb_r[...]) + fb1_r[...])
        out_r[...] = _dot(z1, fw2_r[...]) + fb2_r[...]


def _full(shape):
    return pl.BlockSpec(shape, lambda i: tuple(i * 0 for _ in shape))


def _tc_stage1(cnt0, cnt1, x, w0, w1):
    return pl.pallas_call(
        _stage1_body,
        grid=(NBLK,),
        in_specs=[
            pl.BlockSpec((1, 1, RB), lambda i: (i, i * 0, i * 0)),
            pl.BlockSpec((1, 1, RB), lambda i: (i, i * 0, i * 0)),
            pl.BlockSpec((RB, D), lambda i: (i, i * 0)),
            _full((D, HID)),
            _full((D, HID)),
        ],
        out_specs=[
            pl.BlockSpec((RB, HID), lambda i: (i, i * 0)),
            pl.BlockSpec((RB, HID), lambda i: (i, i * 0)),
        ],
        out_shape=[jax.ShapeDtypeStruct((N, HID), jnp.float32),
                   jax.ShapeDtypeStruct((N, HID), jnp.float32)],
    )(cnt0, cnt1, x, w0, w1)


def _tc_stage2(cnt0, cnt1, s0, s1, u0, u1, wc, bc):
    return pl.pallas_call(
        _stage2_body,
        grid=(NBLK,),
        in_specs=[
            pl.BlockSpec((1, 1, RB), lambda i: (i, i * 0, i * 0)),
            pl.BlockSpec((1, 1, RB), lambda i: (i, i * 0, i * 0)),
            pl.BlockSpec((RB, HID), lambda i: (i, i * 0)),
            pl.BlockSpec((RB, HID), lambda i: (i, i * 0)),
            pl.BlockSpec((RB, HID), lambda i: (i, i * 0)),
            pl.BlockSpec((RB, HID), lambda i: (i, i * 0)),
            _full((HID, HID)),
            _full((1, HID)),
        ],
        out_specs=pl.BlockSpec((RB, HID), lambda i: (i, i * 0)),
        out_shape=jax.ShapeDtypeStruct((N, HID), jnp.float32),
    )(cnt0, cnt1, s0, s1, u0, u1, wc, bc)


def _tc_pool_head(batch3, h, ogt2, ow1, ob1, ow2, ob2, fa, fb, fb1, fw2, fb2):
    return pl.pallas_call(
        _pool_body,
        grid=(NBLK,),
        in_specs=[
            pl.BlockSpec((1, 1, RB), lambda i: (i, i * 0, i * 0)),
            pl.BlockSpec((RB, HID), lambda i: (i, i * 0)),
            _full((G, 1)),
            _full((1, 20)),
            _full((1, 20)),
            _full((20, 10)),
            _full((1, 10)),
            _full((HID, 128)),
            _full((10, 128)),
            _full((1, 128)),
            _full((128, 1)),
            _full((1, 1)),
        ],
        out_specs=pl.BlockSpec((G, 1), lambda i: (i * 0, i * 0)),
        out_shape=jax.ShapeDtypeStruct((G, 1), jnp.float32),
        scratch_shapes=[
            pltpu.VMEM((G, HID), jnp.float32),
            pltpu.VMEM((1, G), jnp.float32),
        ],
    )(batch3, h, ogt2, ow1, ob1, ow2, ob2, fa, fb, fb1, fw2, fb2)


def kernel(x, edge_index_d0, edge_index_d1, batch, ogt,
           W0_d0, W0_d1, Wc0, bc0, W1_d0, W1_d1, Wc1, bc1,
           ogt_W1, ogt_b1, ogt_W2, ogt_b2,
           fc_W1, fc_b1, fc_W2, fc_b2):
    e0 = edge_index_d0.astype(jnp.int32)
    e1 = edge_index_d1.astype(jnp.int32)
    src_both = jnp.concatenate([e0[0], e1[0]])
    dst_both = jnp.concatenate([e0[1], e1[1]])

    zdeg = jnp.zeros((128, 128), jnp.float32)
    zrows = jnp.zeros((ZR, HID // 2), jnp.float32)

    cnt = _sc_degrees(dst_both, zdeg).reshape(2, 16384)[:, :N]
    cnt0 = cnt[0].reshape(NBLK, 1, RB)
    cnt1 = cnt[1].reshape(NBLK, 1, RB)

    u0, u1 = _tc_stage1(cnt0, cnt1, x, W0_d0, W0_d1)
    s0, s1 = _sc_scatter(u0, u1, src_both, dst_both, zrows)
    h = _tc_stage2(cnt0, cnt1, s0, s1, u0, u1, Wc0, bc0.reshape(1, HID))

    u0, u1 = _tc_stage1(cnt0, cnt1, h, W1_d0, W1_d1)
    s0, s1 = _sc_scatter(u0, u1, src_both, dst_both, zrows)
    h = _tc_stage2(cnt0, cnt1, s0, s1, u0, u1, Wc1, bc1.reshape(1, HID))

    batch3 = batch.astype(jnp.int32).reshape(NBLK, 1, RB)
    return _tc_pool_head(batch3, h, ogt[:, None].astype(jnp.float32),
                         ogt_W1, ogt_b1.reshape(1, 20),
                         ogt_W2, ogt_b2.reshape(1, 10),
                         fc_W1[:HID], fc_W1[HID:], fc_b1.reshape(1, 128),
                         fc_W2, fc_b2.reshape(1, 1))


# speedup vs baseline: 7.6178x; 1.9476x over previous
import functools

import jax
import jax.numpy as jnp
from jax import lax
from jax.experimental import pallas as pl
from jax.experimental.pallas import tpu as pltpu
from jax.experimental.pallas import tpu_sc as plsc

N = 10000
E = 160000
D = 256
HID = 256
G = 64

RB = 400
NBLK = N // RB

CUT = 5200
ACCR = 5504
ZR = ACCR // 16
CH = 80
NCH = (E // 16) // CH


def _lrelu(v):
    return jnp.where(v >= 0, v, 0.01 * v)


def _dot(a, b):
    return lax.dot_general(a, b, (((1,), (0,)), ((), ())),
                           precision=lax.Precision.HIGHEST,
                           preferred_element_type=jnp.float32)


def _dot0(a, b):
    return lax.dot_general(a, b, (((0,), (0,)), ((), ())),
                           precision=lax.Precision.HIGHEST,
                           preferred_element_type=jnp.float32)


def _deg_body(dstb_h, zdeg_h, out_h, dd, deg2, tmp, acc, degs):
    c = lax.axis_index("c")
    s = lax.axis_index("s")

    pltpu.sync_copy(zdeg_h, deg2)

    eoff = c * E + s * (E // 16)
    ones16 = jnp.ones((16,), jnp.float32)

    def chunk(k, carry):
        pltpu.sync_copy(dstb_h.at[pl.ds(eoff + CH * k, CH)], dd)
        for j in range(CH // 16):
            dv = dd[pl.ds(16 * j, 16)]
            row = lax.shift_right_logical(dv, jnp.int32(7))
            col = jnp.bitwise_and(dv, jnp.int32(127))
            plsc.addupdate_scatter(deg2, [row, col], ones16)
        return carry

    lax.fori_loop(jnp.int32(0), jnp.int32(NCH), chunk, jnp.int32(0))

    pltpu.sync_copy(deg2, degs.at[s])
    plsc.subcore_barrier()

    for i in range(16):
        pltpu.sync_copy(degs.at[jnp.int32(i), pl.ds(8 * s, 8)],
                        tmp.at[jnp.int32(i)])
    for r in range(8):
        for j in range(8):
            v = tmp[jnp.int32(0), jnp.int32(r), pl.ds(16 * j, 16)]
            for i in range(1, 16):
                v = v + tmp[jnp.int32(i), jnp.int32(r), pl.ds(16 * j, 16)]
            acc[jnp.int32(r), pl.ds(16 * j, 16)] = v
    pltpu.sync_copy(acc, out_h.at[c, s])


def _scat_body(u0_h, u1_h, srcb_h, dstb_h, z_h, o0_h, o1_h,
               sidx, didx, dloc, bufa, bufb, acca, accb, s_ix, s_g,
               ssc0, ssc1):
    c = lax.axis_index("c")
    s = lax.axis_index("s")
    i0 = jnp.int32(0)
    base = c * CUT
    size = CUT - 400 * c
    iota16 = lax.broadcasted_iota(jnp.int32, (16,), 0)
    trash = size + s * 16 + iota16
    HH = HID // 2

    for d, (u_h, o_h) in enumerate(((u0_h, o0_h), (u1_h, o1_h))):
        pltpu.sync_copy(z_h, acca.at[pl.ds(ZR * s, ZR)])
        pltpu.sync_copy(z_h, accb.at[pl.ds(ZR * s, ZR)])
        plsc.subcore_barrier()

        eoff = d * E + s * (E // 16)
        i1 = jnp.int32(1)

        def idx_start(k, b):
            pltpu.async_copy(srcb_h.at[pl.ds(eoff + CH * k, CH)],
                             sidx.at[b], s_ix)
            pltpu.async_copy(dstb_h.at[pl.ds(eoff + CH * k, CH)],
                             didx.at[b], s_ix)

        def idx_wait(b):
            pltpu.make_async_copy(srcb_h.at[pl.ds(eoff, CH)],
                                  sidx.at[b], s_ix).wait()
            pltpu.make_async_copy(dstb_h.at[pl.ds(eoff, CH)],
                                  didx.at[b], s_ix).wait()

        def compute(b):
            for j in range(CH // 16):
                dv = didx[b, pl.ds(16 * j, 16)]
                inr = jnp.logical_and(dv >= base, dv < base + size)
                dloc[b, pl.ds(16 * j, 16)] = jnp.where(inr, dv - base, trash)

        def gather_start(b):
            pltpu.async_copy(u_h.at[sidx.at[b], pl.ds(0, HH)],
                             bufa.at[b], s_g)
            pltpu.async_copy(u_h.at[sidx.at[b], pl.ds(HH, HH)],
                             bufb.at[b], s_g)

        def gather_wait(b):
            pltpu.make_async_copy(u_h.at[sidx.at[b], pl.ds(0, HH)],
                                  bufa.at[b], s_g).wait()
            pltpu.make_async_copy(u_h.at[sidx.at[b], pl.ds(HH, HH)],
                                  bufb.at[b], s_g).wait()

        def scat_start(b, sem):
            pltpu.async_copy(bufa.at[b], acca.at[dloc.at[b]], sem, add=True)
            pltpu.async_copy(bufb.at[b], accb.at[dloc.at[b]], sem, add=True)

        def scat_wait(b, sem):
            pltpu.make_async_copy(bufa.at[b], acca.at[dloc.at[b]], sem).wait()
            pltpu.make_async_copy(bufb.at[b], accb.at[dloc.at[b]], sem).wait()

        idx_start(jnp.int32(0), i0)

        def pipe(t, carry):
            k0 = 2 * t
            idx_wait(i0)

            @pl.when(t > 0)
            def _():
                scat_wait(i0, ssc0)

            compute(i0)
            gather_start(i0)
            idx_start(k0 + 1, i1)
            gather_wait(i0)
            scat_start(i0, ssc0)
            idx_wait(i1)

            @pl.when(t > 0)
            def _():
                scat_wait(i1, ssc1)

            compute(i1)
            gather_start(i1)

            @pl.when(t < (NCH - 1) // 2 - 1)
            def _():
                idx_start(k0 + 2, i0)

            gather_wait(i1)
            scat_start(i1, ssc1)
            return carry

        lax.fori_loop(jnp.int32(0), jnp.int32((NCH - 1) // 2), pipe,
                      jnp.int32(0))
        scat_wait(i0, ssc0)
        idx_start(jnp.int32(NCH - 1), i0)
        idx_wait(i0)
        compute(i0)
        gather_start(i0)
        gather_wait(i0)
        scat_start(i0, ssc0)
        scat_wait(i0, ssc0)
        scat_wait(i1, ssc1)
        plsc.subcore_barrier()

        @pl.when(jnp.logical_and(c == 0, s < 13))
        def _():
            pltpu.sync_copy(acca.at[pl.ds(400 * s, 400)],
                            o_h.at[pl.ds(400 * s, 400), pl.ds(0, HH)])
            pltpu.sync_copy(accb.at[pl.ds(400 * s, 400)],
                            o_h.at[pl.ds(400 * s, 400), pl.ds(HH, HH)])

        @pl.when(jnp.logical_and(c == 1, s < 12))
        def _():
            pltpu.sync_copy(acca.at[pl.ds(400 * s, 400)],
                            o_h.at[pl.ds(CUT + 400 * s, 400), pl.ds(0, HH)])
            pltpu.sync_copy(accb.at[pl.ds(400 * s, 400)],
                            o_h.at[pl.ds(CUT + 400 * s, 400), pl.ds(HH, HH)])

        plsc.subcore_barrier()


def _sc_degrees(dst_both, zdeg):
    mesh = plsc.VectorSubcoreMesh(core_axis_name="c", subcore_axis_name="s")
    f = functools.partial(
        pl.kernel, mesh=mesh,
        compiler_params=pltpu.CompilerParams(needs_layout_passes=False),
        out_type=jax.ShapeDtypeStruct((2, 16, 8, 128), jnp.float32),
        scratch_types=[
            pltpu.VMEM((CH,), jnp.int32),
            pltpu.VMEM((128, 128), jnp.float32),
            pltpu.VMEM((16, 8, 128), jnp.float32),
            pltpu.VMEM((8, 128), jnp.float32),
            pltpu.VMEM_SHARED((16, 128, 128), jnp.float32),
        ],
    )(_deg_body)
    return f(dst_both, zdeg)


def _sc_scatter(u0, u1, src_both, dst_both, zrows):
    mesh = plsc.VectorSubcoreMesh(core_axis_name="c", subcore_axis_name="s")
    f = functools.partial(
        pl.kernel, mesh=mesh,
        compiler_params=pltpu.CompilerParams(needs_layout_passes=False),
        out_type=(jax.ShapeDtypeStruct((N, HID), jnp.float32),
                  jax.ShapeDtypeStruct((N, HID), jnp.float32)),
        scratch_types=[
            pltpu.VMEM((2, CH), jnp.int32),
            pltpu.VMEM((2, CH), jnp.int32),
            pltpu.VMEM((2, CH), jnp.int32),
            pltpu.VMEM((2, CH, HID // 2), jnp.float32),
            pltpu.VMEM((2, CH, HID // 2), jnp.float32),
            pltpu.VMEM_SHARED((ACCR, HID // 2), jnp.float32),
            pltpu.VMEM_SHARED((ACCR, HID // 2), jnp.float32),
            pltpu.SemaphoreType.DMA,
            pltpu.SemaphoreType.DMA,
            pltpu.SemaphoreType.DMA,
            pltpu.SemaphoreType.DMA,
        ],
    )(_scat_body)
    return f(u0, u1, src_both, dst_both, zrows)


def _stage1_body(cnt0_r, cnt1_r, x_r, w0_r, w1_r, u0_r, u1_r):
    xb = x_r[...]
    d0 = lax.rsqrt(cnt0_r[0, 0, :] + 1.0)
    d1 = lax.rsqrt(cnt1_r[0, 0, :] + 1.0)
    u0_r[...] = _dot(xb, w0_r[...]) * d0[:, None]
    u1_r[...] = _dot(xb, w1_r[...]) * d1[:, None]


def _stage2_body(cnt0_r, cnt1_r, s0_r, s1_r, u0_r, u1_r, wc_r, bc_r, h_r):
    d0 = lax.rsqrt(cnt0_r[0, 0, :] + 1.0)
    d1 = lax.rsqrt(cnt1_r[0, 0, :] + 1.0)
    m = 0.5 * (d0[:, None] * (s0_r[...] + u0_r[...])
               + d1[:, None] * (s1_r[...] + u1_r[...]))
    h_r[...] = _lrelu(_dot(m, wc_r[...]) + bc_r[...])


def _pool_body(batch_r, h_r, ogt_r, ow1_r, ob1_r, ow2_r, ob2_r,
               fa_r, fb_r, fb1_r, fw2_r, fb2_r, out_r, pooled, cnts):
    i = pl.program_id(0)

    @pl.when(i == 0)
    def _():
        pooled[...] = jnp.zeros((G, HID), jnp.float32)
        cnts[...] = jnp.zeros((1, G), jnp.float32)

    b = batch_r[0, 0, :]
    oh = (b[:, None] == lax.broadcasted_iota(jnp.int32, (RB, G), 1)
          ).astype(jnp.float32)
    pooled[...] += _dot0(oh, h_r[...])
    cnts[...] += jnp.sum(oh, axis=0)[None, :]

    @pl.when(i == NBLK - 1)
    def _():
        cc = jnp.maximum(cnts[0, :], 1.0)
        gemb = pooled[...] / cc[:, None]
        o1 = _lrelu(_dot(ogt_r[...], ow1_r[...]) + ob1_r[...])
        o2 = _lrelu(_dot(o1, ow2_r[...]) + ob2_r[...])
        z1 = _lrelu(_dot(gemb, fa_r[...]) + _dot(o2, fb_r[...]) + fb1_r[...])
        out_r[...] = _dot(z1, fw2_r[...]) + fb2_r[...]


def _full(shape):
    return pl.BlockSpec(shape, lambda i: tuple(i * 0 for _ in shape))


def _tc_stage1(cnt0, cnt1, x, w0, w1):
    return pl.pallas_call(
        _stage1_body,
        grid=(NBLK,),
        in_specs=[
            pl.BlockSpec((1, 1, RB), lambda i: (i, i * 0, i * 0)),
            pl.BlockSpec((1, 1, RB), lambda i: (i, i * 0, i * 0)),
            pl.BlockSpec((RB, D), lambda i: (i, i * 0)),
            _full((D, HID)),
            _full((D, HID)),
        ],
        out_specs=[
            pl.BlockSpec((RB, HID), lambda i: (i, i * 0)),
            pl.BlockSpec((RB, HID), lambda i: (i, i * 0)),
        ],
        out_shape=[jax.ShapeDtypeStruct((N, HID), jnp.float32),
                   jax.ShapeDtypeStruct((N, HID), jnp.float32)],
    )(cnt0, cnt1, x, w0, w1)


def _tc_stage2(cnt0, cnt1, s0, s1, u0, u1, wc, bc):
    return pl.pallas_call(
        _stage2_body,
        grid=(NBLK,),
        in_specs=[
            pl.BlockSpec((1, 1, RB), lambda i: (i, i * 0, i * 0)),
            pl.BlockSpec((1, 1, RB), lambda i: (i, i * 0, i * 0)),
            pl.BlockSpec((RB, HID), lambda i: (i, i * 0)),
            pl.BlockSpec((RB, HID), lambda i: (i, i * 0)),
            pl.BlockSpec((RB, HID), lambda i: (i, i * 0)),
            pl.BlockSpec((RB, HID), lambda i: (i, i * 0)),
            _full((HID, HID)),
            _full((1, HID)),
        ],
        out_specs=pl.BlockSpec((RB, HID), lambda i: (i, i * 0)),
        out_shape=jax.ShapeDtypeStruct((N, HID), jnp.float32),
    )(cnt0, cnt1, s0, s1, u0, u1, wc, bc)


def _tc_pool_head(batch3, h, ogt2, ow1, ob1, ow2, ob2, fa, fb, fb1, fw2, fb2):
    return pl.pallas_call(
        _pool_body,
        grid=(NBLK,),
        in_specs=[
            pl.BlockSpec((1, 1, RB), lambda i: (i, i * 0, i * 0)),
            pl.BlockSpec((RB, HID), lambda i: (i, i * 0)),
            _full((G, 1)),
            _full((1, 20)),
            _full((1, 20)),
            _full((20, 10)),
            _full((1, 10)),
            _full((HID, 128)),
            _full((10, 128)),
            _full((1, 128)),
            _full((128, 1)),
            _full((1, 1)),
        ],
        out_specs=pl.BlockSpec((G, 1), lambda i: (i * 0, i * 0)),
        out_shape=jax.ShapeDtypeStruct((G, 1), jnp.float32),
        scratch_shapes=[
            pltpu.VMEM((G, HID), jnp.float32),
            pltpu.VMEM((1, G), jnp.float32),
        ],
    )(batch3, h, ogt2, ow1, ob1, ow2, ob2, fa, fb, fb1, fw2, fb2)


def kernel(x, edge_index_d0, edge_index_d1, batch, ogt,
           W0_d0, W0_d1, Wc0, bc0, W1_d0, W1_d1, Wc1, bc1,
           ogt_W1, ogt_b1, ogt_W2, ogt_b2,
           fc_W1, fc_b1, fc_W2, fc_b2):
    e0 = edge_index_d0.astype(jnp.int32)
    e1 = edge_index_d1.astype(jnp.int32)
    src_both = jnp.concatenate([e0[0], e1[0]])
    dst_both = jnp.concatenate([e0[1], e1[1]])

    zdeg = jnp.zeros((128, 128), jnp.float32)
    zrows = jnp.zeros((ZR, HID // 2), jnp.float32)

    cnt = _sc_degrees(dst_both, zdeg).reshape(2, 16384)[:, :N]
    cnt0 = cnt[0].reshape(NBLK, 1, RB)
    cnt1 = cnt[1].reshape(NBLK, 1, RB)

    u0, u1 = _tc_stage1(cnt0, cnt1, x, W0_d0, W0_d1)
    s0, s1 = _sc_scatter(u0, u1, src_both, dst_both, zrows)
    h = _tc_stage2(cnt0, cnt1, s0, s1, u0, u1, Wc0, bc0.reshape(1, HID))

    u0, u1 = _tc_stage1(cnt0, cnt1, h, W1_d0, W1_d1)
    s0, s1 = _sc_scatter(u0, u1, src_both, dst_both, zrows)
    h = _tc_stage2(cnt0, cnt1, s0, s1, u0, u1, Wc1, bc1.reshape(1, HID))

    batch3 = batch.astype(jnp.int32).reshape(NBLK, 1, RB)
    return _tc_pool_head(batch3, h, ogt[:, None].astype(jnp.float32),
                         ogt_W1, ogt_b1.reshape(1, 20),
                         ogt_W2, ogt_b2.reshape(1, 10),
                         fc_W1[:HID], fc_W1[HID:], fc_b1.reshape(1, 128),
                         fc_W2, fc_b2.reshape(1, 1))


# speedup vs baseline: 11.3249x; 1.4866x over previous
import functools

import jax
import jax.numpy as jnp
from jax import lax
from jax.experimental import pallas as pl
from jax.experimental.pallas import tpu as pltpu
from jax.experimental.pallas import tpu_sc as plsc

N = 10000
E = 160000
D = 256
HID = 256
G = 64

RB = 400
NBLK = N // RB

CUT = 5200
ACCR = 5504
ZR = ACCR // 16
CH = 128
DCH = 80
DNCH = (E // 16) // DCH
NFULL = (E // 16) // CH
NPAIR = NFULL // 2
CHT = (E // 16) - CH * NFULL
ACC2 = 10240


def _lrelu(v):
    return jnp.where(v >= 0, v, 0.01 * v)


def _dot(a, b):
    return lax.dot_general(a, b, (((1,), (0,)), ((), ())),
                           precision=lax.Precision.HIGHEST,
                           preferred_element_type=jnp.float32)


def _dot0(a, b):
    return lax.dot_general(a, b, (((0,), (0,)), ((), ())),
                           precision=lax.Precision.HIGHEST,
                           preferred_element_type=jnp.float32)


def _deg_body(dstb_h, zdeg_h, out_h, dd, deg2, tmp, acc, degs):
    c = lax.axis_index("c")
    s = lax.axis_index("s")

    pltpu.sync_copy(zdeg_h, deg2)

    eoff = c * E + s * (E // 16)
    ones16 = jnp.ones((16,), jnp.float32)

    def chunk(k, carry):
        pltpu.sync_copy(dstb_h.at[pl.ds(eoff + DCH * k, DCH)], dd)
        for j in range(DCH // 16):
            dv = dd[pl.ds(16 * j, 16)]
            row = lax.shift_right_logical(dv, jnp.int32(7))
            col = jnp.bitwise_and(dv, jnp.int32(127))
            plsc.addupdate_scatter(deg2, [row, col], ones16)
        return carry

    lax.fori_loop(jnp.int32(0), jnp.int32(DNCH), chunk, jnp.int32(0))

    pltpu.sync_copy(deg2, degs.at[s])
    plsc.subcore_barrier()

    for i in range(16):
        pltpu.sync_copy(degs.at[jnp.int32(i), pl.ds(8 * s, 8)],
                        tmp.at[jnp.int32(i)])
    for r in range(8):
        for j in range(8):
            v = tmp[jnp.int32(0), jnp.int32(r), pl.ds(16 * j, 16)]
            for i in range(1, 16):
                v = v + tmp[jnp.int32(i), jnp.int32(r), pl.ds(16 * j, 16)]
            acc[jnp.int32(r), pl.ds(16 * j, 16)] = v
    pltpu.sync_copy(acc, out_h.at[c, s])


def _scat_body(u0a_h, u0b_h, u1a_h, u1b_h, srcb_h, dstb_h, z_h,
               o0_h, o1_h, sidx, didx, buf, acc, s_ix, s_g, ssc0, ssc1):
    c = lax.axis_index("c")
    s = lax.axis_index("s")
    i0 = jnp.int32(0)
    i1 = jnp.int32(1)
    HH = HID // 2

    def run_dim(u_h, eoff):
        def idx_start(k, b):
            pltpu.async_copy(srcb_h.at[pl.ds(eoff + CH * k, CH)],
                             sidx.at[b], s_ix)
            pltpu.async_copy(dstb_h.at[pl.ds(eoff + CH * k, CH)],
                             didx.at[b], s_ix)

        def idx_wait(b):
            pltpu.make_async_copy(srcb_h.at[pl.ds(eoff, CH)],
                                  sidx.at[b], s_ix).wait()
            pltpu.make_async_copy(dstb_h.at[pl.ds(eoff, CH)],
                                  didx.at[b], s_ix).wait()

        def gather_start(b):
            pltpu.async_copy(u_h.at[sidx.at[b]], buf.at[b], s_g)

        def gather_wait(b):
            pltpu.make_async_copy(u_h.at[sidx.at[b]], buf.at[b],
                                  s_g).wait()

        def scat_start(b, sem):
            pltpu.async_copy(buf.at[b], acc.at[didx.at[b]], sem, add=True)

        def scat_wait(b, sem):
            pltpu.make_async_copy(buf.at[b], acc.at[didx.at[b]],
                                  sem).wait()

        idx_start(jnp.int32(0), i0)

        def pipe(t, carry):
            k0 = 2 * t
            idx_wait(i0)

            @pl.when(t > 0)
            def _():
                scat_wait(i0, ssc0)

            gather_start(i0)
            idx_start(k0 + 1, i1)
            gather_wait(i0)
            scat_start(i0, ssc0)
            idx_wait(i1)

            @pl.when(t > 0)
            def _():
                scat_wait(i1, ssc1)

            gather_start(i1)

            @pl.when(t < NPAIR - 1)
            def _():
                idx_start(k0 + 2, i0)

            gather_wait(i1)
            scat_start(i1, ssc1)
            return carry

        lax.fori_loop(jnp.int32(0), jnp.int32(NPAIR), pipe, jnp.int32(0))
        scat_wait(i0, ssc0)
        scat_wait(i1, ssc1)

        toff = eoff + CH * NFULL
        pltpu.async_copy(srcb_h.at[pl.ds(toff, CHT)],
                         sidx.at[i0, pl.ds(0, CHT)], s_ix)
        pltpu.async_copy(dstb_h.at[pl.ds(toff, CHT)],
                         didx.at[i0, pl.ds(0, CHT)], s_ix)
        pltpu.make_async_copy(srcb_h.at[pl.ds(toff, CHT)],
                              sidx.at[i0, pl.ds(0, CHT)], s_ix).wait()
        pltpu.make_async_copy(dstb_h.at[pl.ds(toff, CHT)],
                              didx.at[i0, pl.ds(0, CHT)], s_ix).wait()
        pltpu.sync_copy(u_h.at[sidx.at[i0, pl.ds(0, CHT)]],
                        buf.at[i0, pl.ds(0, CHT)])
        pltpu.sync_copy(buf.at[i0, pl.ds(0, CHT)],
                        acc.at[didx.at[i0, pl.ds(0, CHT)]], add=True)

    for d, (ua_h, ub_h, o_h) in enumerate(((u0a_h, u0b_h, o0_h),
                                           (u1a_h, u1b_h, o1_h))):
        pltpu.sync_copy(z_h, acc.at[pl.ds(640 * s, 640)])
        plsc.subcore_barrier()

        eoff = d * E + s * (E // 16)

        @pl.when(c == 0)
        def _():
            run_dim(ua_h, eoff)

        @pl.when(c == 1)
        def _():
            run_dim(ub_h, eoff)

        plsc.subcore_barrier()

        @pl.when(jnp.logical_and(c == 0, s < 15))
        def _():
            pltpu.sync_copy(acc.at[pl.ds(640 * s, 640)],
                            o_h.at[pl.ds(640 * s, 640), pl.ds(0, HH)])

        @pl.when(jnp.logical_and(c == 0, s == 15))
        def _():
            pltpu.sync_copy(acc.at[pl.ds(9600, 400)],
                            o_h.at[pl.ds(9600, 400), pl.ds(0, HH)])

        @pl.when(jnp.logical_and(c == 1, s < 15))
        def _():
            pltpu.sync_copy(acc.at[pl.ds(640 * s, 640)],
                            o_h.at[pl.ds(640 * s, 640), pl.ds(HH, HH)])

        @pl.when(jnp.logical_and(c == 1, s == 15))
        def _():
            pltpu.sync_copy(acc.at[pl.ds(9600, 400)],
                            o_h.at[pl.ds(9600, 400), pl.ds(HH, HH)])

        plsc.subcore_barrier()


def _sc_degrees(dst_both, zdeg):
    mesh = plsc.VectorSubcoreMesh(core_axis_name="c", subcore_axis_name="s")
    f = functools.partial(
        pl.kernel, mesh=mesh,
        compiler_params=pltpu.CompilerParams(needs_layout_passes=False),
        out_type=jax.ShapeDtypeStruct((2, 16, 8, 128), jnp.float32),
        scratch_types=[
            pltpu.VMEM((DCH,), jnp.int32),
            pltpu.VMEM((128, 128), jnp.float32),
            pltpu.VMEM((16, 8, 128), jnp.float32),
            pltpu.VMEM((8, 128), jnp.float32),
            pltpu.VMEM_SHARED((16, 128, 128), jnp.float32),
        ],
    )(_deg_body)
    return f(dst_both, zdeg)


def _sc_scatter(u0a, u0b, u1a, u1b, src_both, dst_both, zrows):
    mesh = plsc.VectorSubcoreMesh(core_axis_name="c", subcore_axis_name="s")
    f = functools.partial(
        pl.kernel, mesh=mesh,
        out_type=(jax.ShapeDtypeStruct((N, HID), jnp.float32),
                  jax.ShapeDtypeStruct((N, HID), jnp.float32)),
        scratch_types=[
            pltpu.VMEM((2, CH), jnp.int32),
            pltpu.VMEM((2, CH), jnp.int32),
            pltpu.VMEM((2, CH, HID // 2), jnp.float32),
            pltpu.VMEM_SHARED((ACC2, HID // 2), jnp.float32),
            pltpu.SemaphoreType.DMA,
            pltpu.SemaphoreType.DMA,
            pltpu.SemaphoreType.DMA,
            pltpu.SemaphoreType.DMA,
        ],
    )(_scat_body)
    return f(u0a, u0b, u1a, u1b, src_both, dst_both, zrows)


def _stage1_body(cnt0_r, cnt1_r, x_r, w0_r, w1_r,
                 u0a_r, u0b_r, u1a_r, u1b_r, u0_r, u1_r):
    xb = x_r[...]
    d0 = lax.rsqrt(cnt0_r[0, 0, :] + 1.0)
    d1 = lax.rsqrt(cnt1_r[0, 0, :] + 1.0)
    HH = HID // 2
    u0 = _dot(xb, w0_r[...]) * d0[:, None]
    u1 = _dot(xb, w1_r[...]) * d1[:, None]
    u0a_r[...] = u0[:, :HH]
    u0b_r[...] = u0[:, HH:]
    u1a_r[...] = u1[:, :HH]
    u1b_r[...] = u1[:, HH:]
    u0_r[...] = u0
    u1_r[...] = u1


def _stage2_body(cnt0_r, cnt1_r, s0_r, s1_r, u0_r, u1_r, wc_r, bc_r, h_r):
    d0 = lax.rsqrt(cnt0_r[0, 0, :] + 1.0)
    d1 = lax.rsqrt(cnt1_r[0, 0, :] + 1.0)
    m = 0.5 * (d0[:, None] * (s0_r[...] + u0_r[...])
               + d1[:, None] * (s1_r[...] + u1_r[...]))
    h_r[...] = _lrelu(_dot(m, wc_r[...]) + bc_r[...])


def _pool_body(batch_r, h_r, ogt_r, ow1_r, ob1_r, ow2_r, ob2_r,
               fa_r, fb_r, fb1_r, fw2_r, fb2_r, out_r, pooled, cnts):
    i = pl.program_id(0)

    @pl.when(i == 0)
    def _():
        pooled[...] = jnp.zeros((G, HID), jnp.float32)
        cnts[...] = jnp.zeros((1, G), jnp.float32)

    b = batch_r[0, 0, :]
    oh = (b[:, None] == lax.broadcasted_iota(jnp.int32, (RB, G), 1)
          ).astype(jnp.float32)
    pooled[...] += _dot0(oh, h_r[...])
    cnts[...] += jnp.sum(oh, axis=0)[None, :]

    @pl.when(i == NBLK - 1)
    def _():
        cc = jnp.maximum(cnts[0, :], 1.0)
        gemb = pooled[...] / cc[:, None]
        o1 = _lrelu(_dot(ogt_r[...], ow1_r[...]) + ob1_r[...])
        o2 = _lrelu(_dot(o1, ow2_r[...]) + ob2_r[...])
        z1 = _lrelu(_dot(gemb, fa_r[...]) + _dot(o2, fb_r[...]) + fb1_r[...])
        out_r[...] = _dot(z1, fw2_r[...]) + fb2_r[...]


def _full(shape):
    return pl.BlockSpec(shape, lambda i: tuple(i * 0 for _ in shape))


def _tc_stage1(cnt0, cnt1, x, w0, w1):
    return pl.pallas_call(
        _stage1_body,
        grid=(NBLK,),
        in_specs=[
            pl.BlockSpec((1, 1, RB), lambda i: (i, i * 0, i * 0)),
            pl.BlockSpec((1, 1, RB), lambda i: (i, i * 0, i * 0)),
            pl.BlockSpec((RB, D), lambda i: (i, i * 0)),
            _full((D, HID)),
            _full((D, HID)),
        ],
        out_specs=[
            pl.BlockSpec((RB, HID // 2), lambda i: (i, i * 0)),
            pl.BlockSpec((RB, HID // 2), lambda i: (i, i * 0)),
            pl.BlockSpec((RB, HID // 2), lambda i: (i, i * 0)),
            pl.BlockSpec((RB, HID // 2), lambda i: (i, i * 0)),
            pl.BlockSpec((RB, HID), lambda i: (i, i * 0)),
            pl.BlockSpec((RB, HID), lambda i: (i, i * 0)),
        ],
        out_shape=[jax.ShapeDtypeStruct((N, HID // 2), jnp.float32),
                   jax.ShapeDtypeStruct((N, HID // 2), jnp.float32),
                   jax.ShapeDtypeStruct((N, HID // 2), jnp.float32),
                   jax.ShapeDtypeStruct((N, HID // 2), jnp.float32),
                   jax.ShapeDtypeStruct((N, HID), jnp.float32),
                   jax.ShapeDtypeStruct((N, HID), jnp.float32)],
    )(cnt0, cnt1, x, w0, w1)


def _tc_stage2(cnt0, cnt1, s0, s1, u0, u1, wc, bc):
    return pl.pallas_call(
        _stage2_body,
        grid=(NBLK,),
        in_specs=[
            pl.BlockSpec((1, 1, RB), lambda i: (i, i * 0, i * 0)),
            pl.BlockSpec((1, 1, RB), lambda i: (i, i * 0, i * 0)),
            pl.BlockSpec((RB, HID), lambda i: (i, i * 0)),
            pl.BlockSpec((RB, HID), lambda i: (i, i * 0)),
            pl.BlockSpec((RB, HID), lambda i: (i, i * 0)),
            pl.BlockSpec((RB, HID), lambda i: (i, i * 0)),
            _full((HID, HID)),
            _full((1, HID)),
        ],
        out_specs=pl.BlockSpec((RB, HID), lambda i: (i, i * 0)),
        out_shape=jax.ShapeDtypeStruct((N, HID), jnp.float32),
    )(cnt0, cnt1, s0, s1, u0, u1, wc, bc)


def _tc_pool_head(batch3, h, ogt2, ow1, ob1, ow2, ob2, fa, fb, fb1, fw2, fb2):
    return pl.pallas_call(
        _pool_body,
        grid=(NBLK,),
        in_specs=[
            pl.BlockSpec((1, 1, RB), lambda i: (i, i * 0, i * 0)),
            pl.BlockSpec((RB, HID), lambda i: (i, i * 0)),
            _full((G, 1)),
            _full((1, 20)),
            _full((1, 20)),
            _full((20, 10)),
            _full((1, 10)),
            _full((HID, 128)),
            _full((10, 128)),
            _full((1, 128)),
            _full((128, 1)),
            _full((1, 1)),
        ],
        out_specs=pl.BlockSpec((G, 1), lambda i: (i * 0, i * 0)),
        out_shape=jax.ShapeDtypeStruct((G, 1), jnp.float32),
        scratch_shapes=[
            pltpu.VMEM((G, HID), jnp.float32),
            pltpu.VMEM((1, G), jnp.float32),
        ],
    )(batch3, h, ogt2, ow1, ob1, ow2, ob2, fa, fb, fb1, fw2, fb2)


def kernel(x, edge_index_d0, edge_index_d1, batch, ogt,
           W0_d0, W0_d1, Wc0, bc0, W1_d0, W1_d1, Wc1, bc1,
           ogt_W1, ogt_b1, ogt_W2, ogt_b2,
           fc_W1, fc_b1, fc_W2, fc_b2):
    e0 = edge_index_d0.astype(jnp.int32)
    e1 = edge_index_d1.astype(jnp.int32)
    src_both = jnp.concatenate([e0[0], e1[0]])
    dst_both = jnp.concatenate([e0[1], e1[1]])

    zdeg = jnp.zeros((128, 128), jnp.float32)
    zrows = jnp.zeros((640, HID // 2), jnp.float32)

    cnt = _sc_degrees(dst_both, zdeg).reshape(2, 16384)[:, :N]
    cnt0 = cnt[0].reshape(NBLK, 1, RB)
    cnt1 = cnt[1].reshape(NBLK, 1, RB)

    u0a, u0b, u1a, u1b, u0, u1 = _tc_stage1(cnt0, cnt1, x, W0_d0, W0_d1)
    s0, s1 = _sc_scatter(u0a, u0b, u1a, u1b, src_both, dst_both, zrows)
    h = _tc_stage2(cnt0, cnt1, s0, s1, u0, u1, Wc0, bc0.reshape(1, HID))

    u0a, u0b, u1a, u1b, u0, u1 = _tc_stage1(cnt0, cnt1, h, W1_d0, W1_d1)
    s0, s1 = _sc_scatter(u0a, u0b, u1a, u1b, src_both, dst_both, zrows)
    h = _tc_stage2(cnt0, cnt1, s0, s1, u0, u1, Wc1, bc1.reshape(1, HID))

    batch3 = batch.astype(jnp.int32).reshape(NBLK, 1, RB)
    return _tc_pool_head(batch3, h, ogt[:, None].astype(jnp.float32),
                         ogt_W1, ogt_b1.reshape(1, 20),
                         ogt_W2, ogt_b2.reshape(1, 10),
                         fc_W1[:HID], fc_W1[HID:], fc_b1.reshape(1, 128),
                         fc_W2, fc_b2.reshape(1, 1))


# speedup vs baseline: 12.4583x; 1.1001x over previous
import functools

import jax
import jax.numpy as jnp
from jax import lax
from jax.experimental import pallas as pl
from jax.experimental.pallas import tpu as pltpu
from jax.experimental.pallas import tpu_sc as plsc

N = 10000
E = 160000
D = 256
HID = 256
G = 64

RB = 400
NBLK = N // RB

CUT = 5200
ACCR = 5504
ZR = ACCR // 16
CH = 128
DCH = 80
DNCH = (E // 16) // DCH
NFULL = (E // 16) // CH
NPAIR = NFULL // 2
CHT = (E // 16) - CH * NFULL
ACC2 = 10240


def _lrelu(v):
    return jnp.where(v >= 0, v, 0.01 * v)


def _dot(a, b):
    return lax.dot_general(a, b, (((1,), (0,)), ((), ())),
                           preferred_element_type=jnp.float32)


def _dot0(a, b):
    return lax.dot_general(a, b, (((0,), (0,)), ((), ())),
                           preferred_element_type=jnp.float32)


def _deg_body(dstb_h, zdeg_h, out_h, dd, deg2, tmp, acc, degs):
    c = lax.axis_index("c")
    s = lax.axis_index("s")

    pltpu.sync_copy(zdeg_h, deg2)

    eoff = c * E + s * (E // 16)
    ones16 = jnp.ones((16,), jnp.float32)

    def chunk(k, carry):
        pltpu.sync_copy(dstb_h.at[pl.ds(eoff + DCH * k, DCH)], dd)
        for j in range(DCH // 16):
            dv = dd[pl.ds(16 * j, 16)]
            row = lax.shift_right_logical(dv, jnp.int32(7))
            col = jnp.bitwise_and(dv, jnp.int32(127))
            plsc.addupdate_scatter(deg2, [row, col], ones16)
        return carry

    lax.fori_loop(jnp.int32(0), jnp.int32(DNCH), chunk, jnp.int32(0))

    pltpu.sync_copy(deg2, degs.at[s])
    plsc.subcore_barrier()

    for i in range(16):
        pltpu.sync_copy(degs.at[jnp.int32(i), pl.ds(8 * s, 8)],
                        tmp.at[jnp.int32(i)])
    for r in range(8):
        for j in range(8):
            v = tmp[jnp.int32(0), jnp.int32(r), pl.ds(16 * j, 16)]
            for i in range(1, 16):
                v = v + tmp[jnp.int32(i), jnp.int32(r), pl.ds(16 * j, 16)]
            acc[jnp.int32(r), pl.ds(16 * j, 16)] = v
    pltpu.sync_copy(acc, out_h.at[c, s])


def _scat_body(u0a_h, u0b_h, u1a_h, u1b_h, srcb_h, dstb_h, z_h,
               o0_h, o1_h, sidx, didx, buf, acc, s_ix, sg0, sg1,
               ssc0, ssc1):
    c = lax.axis_index("c")
    s = lax.axis_index("s")
    i0 = jnp.int32(0)
    i1 = jnp.int32(1)
    HH = HID // 2

    def run_dim(u_h, eoff):
        def idx_start(k, b):
            pltpu.async_copy(srcb_h.at[pl.ds(eoff + CH * k, CH)],
                             sidx.at[b], s_ix)
            pltpu.async_copy(dstb_h.at[pl.ds(eoff + CH * k, CH)],
                             didx.at[b], s_ix)

        def idx_wait(b):
            pltpu.make_async_copy(srcb_h.at[pl.ds(eoff, CH)],
                                  sidx.at[b], s_ix).wait()
            pltpu.make_async_copy(dstb_h.at[pl.ds(eoff, CH)],
                                  didx.at[b], s_ix).wait()

        def gather_start(b, sem):
            pltpu.async_copy(u_h.at[sidx.at[b]], buf.at[b], sem)

        def gather_wait(b, sem):
            pltpu.make_async_copy(u_h.at[sidx.at[b]], buf.at[b],
                                  sem).wait()

        def scat_start(b, sem):
            pltpu.async_copy(buf.at[b], acc.at[didx.at[b]], sem, add=True)

        def scat_wait(b, sem):
            pltpu.make_async_copy(buf.at[b], acc.at[didx.at[b]],
                                  sem).wait()

        idx_start(jnp.int32(0), i0)

        def pipe(t, carry):
            k0 = 2 * t
            idx_wait(i0)

            @pl.when(t > 0)
            def _():
                scat_wait(i0, ssc0)

            gather_start(i0, sg0)
            idx_start(k0 + 1, i1)
            idx_wait(i1)

            @pl.when(t > 0)
            def _():
                scat_wait(i1, ssc1)

            gather_start(i1, sg1)
            gather_wait(i0, sg0)
            scat_start(i0, ssc0)

            @pl.when(t < NPAIR - 1)
            def _():
                idx_start(k0 + 2, i0)

            gather_wait(i1, sg1)
            scat_start(i1, ssc1)
            return carry

        lax.fori_loop(jnp.int32(0), jnp.int32(NPAIR), pipe, jnp.int32(0))
        scat_wait(i0, ssc0)
        scat_wait(i1, ssc1)

        toff = eoff + CH * NFULL
        pltpu.async_copy(srcb_h.at[pl.ds(toff, CHT)],
                         sidx.at[i0, pl.ds(0, CHT)], s_ix)
        pltpu.async_copy(dstb_h.at[pl.ds(toff, CHT)],
                         didx.at[i0, pl.ds(0, CHT)], s_ix)
        pltpu.make_async_copy(srcb_h.at[pl.ds(toff, CHT)],
                              sidx.at[i0, pl.ds(0, CHT)], s_ix).wait()
        pltpu.make_async_copy(dstb_h.at[pl.ds(toff, CHT)],
                              didx.at[i0, pl.ds(0, CHT)], s_ix).wait()
        pltpu.sync_copy(u_h.at[sidx.at[i0, pl.ds(0, CHT)]],
                        buf.at[i0, pl.ds(0, CHT)])
        pltpu.sync_copy(buf.at[i0, pl.ds(0, CHT)],
                        acc.at[didx.at[i0, pl.ds(0, CHT)]], add=True)

    for d, (ua_h, ub_h, o_h) in enumerate(((u0a_h, u0b_h, o0_h),
                                           (u1a_h, u1b_h, o1_h))):
        pltpu.sync_copy(z_h, acc.at[pl.ds(640 * s, 640)])
        plsc.subcore_barrier()

        eoff = d * E + s * (E // 16)

        @pl.when(c == 0)
        def _():
            run_dim(ua_h, eoff)

        @pl.when(c == 1)
        def _():
            run_dim(ub_h, eoff)

        plsc.subcore_barrier()

        @pl.when(jnp.logical_and(c == 0, s < 15))
        def _():
            pltpu.sync_copy(acc.at[pl.ds(640 * s, 640)],
                            o_h.at[pl.ds(640 * s, 640), pl.ds(0, HH)])

        @pl.when(jnp.logical_and(c == 0, s == 15))
        def _():
            pltpu.sync_copy(acc.at[pl.ds(9600, 400)],
                            o_h.at[pl.ds(9600, 400), pl.ds(0, HH)])

        @pl.when(jnp.logical_and(c == 1, s < 15))
        def _():
            pltpu.sync_copy(acc.at[pl.ds(640 * s, 640)],
                            o_h.at[pl.ds(640 * s, 640), pl.ds(HH, HH)])

        @pl.when(jnp.logical_and(c == 1, s == 15))
        def _():
            pltpu.sync_copy(acc.at[pl.ds(9600, 400)],
                            o_h.at[pl.ds(9600, 400), pl.ds(HH, HH)])

        plsc.subcore_barrier()


def _sc_degrees(dst_both, zdeg):
    mesh = plsc.VectorSubcoreMesh(core_axis_name="c", subcore_axis_name="s")
    f = functools.partial(
        pl.kernel, mesh=mesh,
        compiler_params=pltpu.CompilerParams(needs_layout_passes=False),
        out_type=jax.ShapeDtypeStruct((2, 16, 8, 128), jnp.float32),
        scratch_types=[
            pltpu.VMEM((DCH,), jnp.int32),
            pltpu.VMEM((128, 128), jnp.float32),
            pltpu.VMEM((16, 8, 128), jnp.float32),
            pltpu.VMEM((8, 128), jnp.float32),
            pltpu.VMEM_SHARED((16, 128, 128), jnp.float32),
        ],
    )(_deg_body)
    return f(dst_both, zdeg)


def _sc_scatter(u0a, u0b, u1a, u1b, src_both, dst_both, zrows):
    mesh = plsc.VectorSubcoreMesh(core_axis_name="c", subcore_axis_name="s")
    f = functools.partial(
        pl.kernel, mesh=mesh,
        out_type=(jax.ShapeDtypeStruct((N, HID), jnp.float32),
                  jax.ShapeDtypeStruct((N, HID), jnp.float32)),
        scratch_types=[
            pltpu.VMEM((2, CH), jnp.int32),
            pltpu.VMEM((2, CH), jnp.int32),
            pltpu.VMEM((2, CH, HID // 2), jnp.float32),
            pltpu.VMEM_SHARED((ACC2, HID // 2), jnp.float32),
            pltpu.SemaphoreType.DMA,
            pltpu.SemaphoreType.DMA,
            pltpu.SemaphoreType.DMA,
            pltpu.SemaphoreType.DMA,
            pltpu.SemaphoreType.DMA,
        ],
    )(_scat_body)
    return f(u0a, u0b, u1a, u1b, src_both, dst_both, zrows)


def _stage1_body(cnt0_r, cnt1_r, x_r, w0_r, w1_r,
                 u0a_r, u0b_r, u1a_r, u1b_r, u0_r, u1_r):
    xb = x_r[...]
    d0 = lax.rsqrt(cnt0_r[0, 0, :] + 1.0)
    d1 = lax.rsqrt(cnt1_r[0, 0, :] + 1.0)
    HH = HID // 2
    u0 = _dot(xb, w0_r[...]) * d0[:, None]
    u1 = _dot(xb, w1_r[...]) * d1[:, None]
    u0a_r[...] = u0[:, :HH]
    u0b_r[...] = u0[:, HH:]
    u1a_r[...] = u1[:, :HH]
    u1b_r[...] = u1[:, HH:]
    u0_r[...] = u0
    u1_r[...] = u1


def _stage2_body(cnt0_r, cnt1_r, s0_r, s1_r, u0_r, u1_r, wc_r, bc_r, h_r):
    d0 = lax.rsqrt(cnt0_r[0, 0, :] + 1.0)
    d1 = lax.rsqrt(cnt1_r[0, 0, :] + 1.0)
    m = 0.5 * (d0[:, None] * (s0_r[...] + u0_r[...])
               + d1[:, None] * (s1_r[...] + u1_r[...]))
    h_r[...] = _lrelu(_dot(m, wc_r[...]) + bc_r[...])


def _pool_body(batch_r, h_r, ogt_r, ow1_r, ob1_r, ow2_r, ob2_r,
               fa_r, fb_r, fb1_r, fw2_r, fb2_r, out_r, pooled, cnts):
    i = pl.program_id(0)

    @pl.when(i == 0)
    def _():
        pooled[...] = jnp.zeros((G, HID), jnp.float32)
        cnts[...] = jnp.zeros((1, G), jnp.float32)

    b = batch_r[0, 0, :]
    oh = (b[:, None] == lax.broadcasted_iota(jnp.int32, (RB, G), 1)
          ).astype(jnp.float32)
    pooled[...] += _dot0(oh, h_r[...])
    cnts[...] += jnp.sum(oh, axis=0)[None, :]

    @pl.when(i == NBLK - 1)
    def _():
        cc = jnp.maximum(cnts[0, :], 1.0)
        gemb = pooled[...] / cc[:, None]
        o1 = _lrelu(_dot(ogt_r[...], ow1_r[...]) + ob1_r[...])
        o2 = _lrelu(_dot(o1, ow2_r[...]) + ob2_r[...])
        z1 = _lrelu(_dot(gemb, fa_r[...]) + _dot(o2, fb_r[...]) + fb1_r[...])
        out_r[...] = _dot(z1, fw2_r[...]) + fb2_r[...]


def _full(shape):
    return pl.BlockSpec(shape, lambda i: tuple(i * 0 for _ in shape))


def _tc_stage1(cnt0, cnt1, x, w0, w1):
    return pl.pallas_call(
        _stage1_body,
        grid=(NBLK,),
        in_specs=[
            pl.BlockSpec((1, 1, RB), lambda i: (i, i * 0, i * 0)),
            pl.BlockSpec((1, 1, RB), lambda i: (i, i * 0, i * 0)),
            pl.BlockSpec((RB, D), lambda i: (i, i * 0)),
            _full((D, HID)),
            _full((D, HID)),
        ],
        out_specs=[
            pl.BlockSpec((RB, HID // 2), lambda i: (i, i * 0)),
            pl.BlockSpec((RB, HID // 2), lambda i: (i, i * 0)),
            pl.BlockSpec((RB, HID // 2), lambda i: (i, i * 0)),
            pl.BlockSpec((RB, HID // 2), lambda i: (i, i * 0)),
            pl.BlockSpec((RB, HID), lambda i: (i, i * 0)),
            pl.BlockSpec((RB, HID), lambda i: (i, i * 0)),
        ],
        out_shape=[jax.ShapeDtypeStruct((N, HID // 2), jnp.float32),
                   jax.ShapeDtypeStruct((N, HID // 2), jnp.float32),
                   jax.ShapeDtypeStruct((N, HID // 2), jnp.float32),
                   jax.ShapeDtypeStruct((N, HID // 2), jnp.float32),
                   jax.ShapeDtypeStruct((N, HID), jnp.float32),
                   jax.ShapeDtypeStruct((N, HID), jnp.float32)],
    )(cnt0, cnt1, x, w0, w1)


def _tc_stage2(cnt0, cnt1, s0, s1, u0, u1, wc, bc):
    return pl.pallas_call(
        _stage2_body,
        grid=(NBLK,),
        in_specs=[
            pl.BlockSpec((1, 1, RB), lambda i: (i, i * 0, i * 0)),
            pl.BlockSpec((1, 1, RB), lambda i: (i, i * 0, i * 0)),
            pl.BlockSpec((RB, HID), lambda i: (i, i * 0)),
            pl.BlockSpec((RB, HID), lambda i: (i, i * 0)),
            pl.BlockSpec((RB, HID), lambda i: (i, i * 0)),
            pl.BlockSpec((RB, HID), lambda i: (i, i * 0)),
            _full((HID, HID)),
            _full((1, HID)),
        ],
        out_specs=pl.BlockSpec((RB, HID), lambda i: (i, i * 0)),
        out_shape=jax.ShapeDtypeStruct((N, HID), jnp.float32),
    )(cnt0, cnt1, s0, s1, u0, u1, wc, bc)


def _tc_pool_head(batch3, h, ogt2, ow1, ob1, ow2, ob2, fa, fb, fb1, fw2, fb2):
    return pl.pallas_call(
        _pool_body,
        grid=(NBLK,),
        in_specs=[
            pl.BlockSpec((1, 1, RB), lambda i: (i, i * 0, i * 0)),
            pl.BlockSpec((RB, HID), lambda i: (i, i * 0)),
            _full((G, 1)),
            _full((1, 20)),
            _full((1, 20)),
            _full((20, 10)),
            _full((1, 10)),
            _full((HID, 128)),
            _full((10, 128)),
            _full((1, 128)),
            _full((128, 1)),
            _full((1, 1)),
        ],
        out_specs=pl.BlockSpec((G, 1), lambda i: (i * 0, i * 0)),
        out_shape=jax.ShapeDtypeStruct((G, 1), jnp.float32),
        scratch_shapes=[
            pltpu.VMEM((G, HID), jnp.float32),
            pltpu.VMEM((1, G), jnp.float32),
        ],
    )(batch3, h, ogt2, ow1, ob1, ow2, ob2, fa, fb, fb1, fw2, fb2)


def kernel(x, edge_index_d0, edge_index_d1, batch, ogt,
           W0_d0, W0_d1, Wc0, bc0, W1_d0, W1_d1, Wc1, bc1,
           ogt_W1, ogt_b1, ogt_W2, ogt_b2,
           fc_W1, fc_b1, fc_W2, fc_b2):
    e0 = edge_index_d0.astype(jnp.int32)
    e1 = edge_index_d1.astype(jnp.int32)
    src_both = jnp.concatenate([e0[0], e1[0]])
    dst_both = jnp.concatenate([e0[1], e1[1]])

    zdeg = jnp.zeros((128, 128), jnp.float32)
    zrows = jnp.zeros((640, HID // 2), jnp.float32)

    cnt = _sc_degrees(dst_both, zdeg).reshape(2, 16384)[:, :N]
    cnt0 = cnt[0].reshape(NBLK, 1, RB)
    cnt1 = cnt[1].reshape(NBLK, 1, RB)

    u0a, u0b, u1a, u1b, u0, u1 = _tc_stage1(cnt0, cnt1, x, W0_d0, W0_d1)
    s0, s1 = _sc_scatter(u0a, u0b, u1a, u1b, src_both, dst_both, zrows)
    h = _tc_stage2(cnt0, cnt1, s0, s1, u0, u1, Wc0, bc0.reshape(1, HID))

    u0a, u0b, u1a, u1b, u0, u1 = _tc_stage1(cnt0, cnt1, h, W1_d0, W1_d1)
    s0, s1 = _sc_scatter(u0a, u0b, u1a, u1b, src_both, dst_both, zrows)
    h = _tc_stage2(cnt0, cnt1, s0, s1, u0, u1, Wc1, bc1.reshape(1, HID))

    batch3 = batch.astype(jnp.int32).reshape(NBLK, 1, RB)
    return _tc_pool_head(batch3, h, ogt[:, None].astype(jnp.float32),
                         ogt_W1, ogt_b1.reshape(1, 20),
                         ogt_W2, ogt_b2.reshape(1, 10),
                         fc_W1[:HID], fc_W1[HID:], fc_b1.reshape(1, 128),
                         fc_W2, fc_b2.reshape(1, 1))
